# baseline (speedup 1.0000x reference)
import functools

import jax
import jax.numpy as jnp
from jax import lax
from jax.experimental import pallas as pl
from jax.experimental.pallas import tpu as pltpu
from jax.experimental.pallas import tpu_sc as plsc

_L = 200
_DIM = 64
_PAD = 256
_NSUB = 16
_NCHUNK = _DIM // 16


_SCRATCH = [
    pltpu.VMEM((2 * _PAD,), jnp.int32),
    pltpu.VMEM((16,), jnp.int32),
    pltpu.VMEM((16, 2 * _DIM), jnp.float32),
    pltpu.VMEM((_NSUB * _DIM,), jnp.float32),
    pltpu.VMEM((_DIM,), jnp.float32),
    pltpu.VMEM_SHARED((_NSUB * _DIM,), jnp.float32),
    pltpu.SemaphoreType.DMA,
]


def _emb_agg_body(idx_hbm, table2_hbm, out_hbm,
                  idx_all_v, row_v, rows_v, red_v, vec_v, shared, sem):
    cid = lax.axis_index("c")
    sid = lax.axis_index("s")
    base = cid * _PAD + sid * 16

    pltpu.sync_copy(idx_hbm, idx_all_v)
    iv = idx_all_v[pl.ds(base, 16)]
    row_v[...] = jax.lax.shift_right_logical(iv, 1)
    pv = (iv & 1).astype(jnp.float32)

    pltpu.async_copy(table2_hbm.at[row_v], rows_v, sem).wait()

    acc = [jnp.zeros((16,), jnp.float32) for _ in range(_NCHUNK)]
    for j in range(16):
        valid = jnp.where(sid * 16 + j < _L, jnp.float32(1.0), jnp.float32(0.0))
        whi = pv[j] * valid
        wlo = valid - whi
        for c in range(_NCHUNK):
            acc[c] = (acc[c]
                      + rows_v[j, pl.ds(c * 16, 16)] * wlo
                      + rows_v[j, pl.ds(_DIM + c * 16, 16)] * whi)
    for c in range(_NCHUNK):
        vec_v[pl.ds(c * 16, 16)] = acc[c]

    pltpu.sync_copy(vec_v, shared.at[pl.ds(sid * _DIM, _DIM)])
    plsc.subcore_barrier()

    @pl.when(sid == 0)
    def _reduce():
        pltpu.sync_copy(shared, red_v)
        tot = [jnp.zeros((16,), jnp.float32) for _ in range(_NCHUNK)]
        for r in range(_NSUB):
            for c in range(_NCHUNK):
                tot[c] = tot[c] + red_v[pl.ds(r * _DIM + c * 16, 16)]
        inv = jnp.float32(1.0 / _L)
        for c in range(_NCHUNK):
            vec_v[pl.ds(c * 16, 16)] = tot[c] * inv
        pltpu.sync_copy(vec_v, out_hbm.at[cid])


_emb_agg = pl.kernel(
    _emb_agg_body,
    out_type=jax.ShapeDtypeStruct((2, _DIM), jnp.float32),
    scratch_types=_SCRATCH,
    mesh=plsc.VectorSubcoreMesh(core_axis_name="c", subcore_axis_name="s"),
    compiler_params=pltpu.CompilerParams(needs_layout_passes=False),
)


def kernel(s1, s2, table):
    table2 = jnp.reshape(table, (table.shape[0] // 2, 2 * _DIM))
    pad = jnp.zeros((_PAD - _L,), jnp.int32)
    idx = jnp.concatenate([s1.astype(jnp.int32), pad,
                           s2.astype(jnp.int32), pad])
    return _emb_agg(idx, table2).reshape(2 * _DIM)
